# baseline (speedup 1.0000x reference)
import functools

import jax
import jax.numpy as jnp
from jax import lax
from jax.experimental import pallas as pl
from jax.experimental.pallas import tpu as pltpu
from jax.experimental.pallas import tpu_sc as plsc

B, T, D = 4, 4096, 2048
K = 4
C = 2048
NUM_TOKENS = 8192
HALF = 1024
MODEL_DIM = 2048

ITEMS = B * T
NB = 8
ROWS = NB * K
LANES = 16

NC = 2
NS = 16


def _sc_gather_sum():
    nw = NC * NS
    ipw = ITEMS // nw
    nblk = ipw // NB
    mesh = plsc.VectorSubcoreMesh(
        core_axis_name="c", subcore_axis_name="s", num_cores=NC, num_subcores=NS
    )

    @functools.partial(
        pl.kernel,
        out_type=jax.ShapeDtypeStruct((ITEMS, HALF), jnp.float32),
        mesh=mesh,
        scratch_types=[
            pltpu.VMEM((ipw * K,), jnp.int32),
            pltpu.VMEM((2, ROWS, HALF), jnp.float32),
            pltpu.VMEM((2, NB, HALF), jnp.float32),
            pltpu.SemaphoreType.DMA,
            pltpu.SemaphoreType.DMA,
        ],
    )
    def sc_kernel(idx_hbm, table_hbm, out_hbm, idx_v, rows_v, out_v, gsem, osem):
        wid = lax.axis_index("s") * NC + lax.axis_index("c")
        base_item = wid * ipw

        pltpu.sync_copy(idx_hbm.at[pl.ds(base_item * K, ipw * K)], idx_v)

        def start_gather(blk, buf):
            pltpu.async_copy(
                table_hbm.at[idx_v.at[pl.ds(blk * ROWS, ROWS)]],
                rows_v.at[buf],
                gsem,
            )

        def wait_gather(buf):
            pltpu.make_async_copy(
                table_hbm.at[idx_v.at[pl.ds(0, ROWS)]], rows_v.at[buf], gsem
            ).wait()

        def wait_out(par):
            pltpu.make_async_copy(
                out_v.at[par], out_hbm.at[pl.ds(0, NB)], osem
            ).wait()

        start_gather(0, 0)

        def blk_pair(h, _):
            for par in (0, 1):
                blk = 2 * h + par
                nxt = jnp.minimum(blk + 1, nblk - 1)
                start_gather(nxt, 1 - par)
                wait_gather(par)

                @pl.when(h > 0)
                def _():
                    wait_out(par)

                @plsc.parallel_loop(0, HALF // LANES, 1, unroll=2)
                def _(j):
                    s = pl.ds(j * LANES, LANES)
                    for i in range(NB):
                        r0 = 4 * i
                        out_v[par, i, s] = (
                            (rows_v[par, r0, s] + rows_v[par, r0 + 1, s])
                            + (rows_v[par, r0 + 2, s] + rows_v[par, r0 + 3, s])
                        )
                pltpu.async_copy(
                    out_v.at[par],
                    out_hbm.at[pl.ds(base_item + blk * NB, NB)],
                    osem,
                )
            return 0

        lax.fori_loop(0, nblk // 2, blk_pair, 0)
        wait_gather(0)
        wait_out(0)
        wait_out(1)

    return sc_kernel


_TB = 512


def _fc_body(inst_ids, x_ref, w_ref, b_ref, inst_ref, out_ref):
    x = x_ref[0].astype(jnp.bfloat16)
    w = w_ref[...]
    acc = lax.dot_general(
        x, w, (((1,), (1,)), ((), ())),
        preferred_element_type=jnp.float32,
    )
    out_ref[0] = acc + b_ref[...] + inst_ref[0]


def _tc_fc(x, w, b, inst_ids, inst_table):
    grid_spec = pltpu.PrefetchScalarGridSpec(
        num_scalar_prefetch=1,
        grid=(B, T // _TB),
        in_specs=[
            pl.BlockSpec((1, _TB, D), lambda bi, ti, ids: (bi, ti, 0)),
            pl.BlockSpec((HALF, D), lambda bi, ti, ids: (0, 0)),
            pl.BlockSpec((1, HALF), lambda bi, ti, ids: (0, 0)),
            pl.BlockSpec((1, 1, HALF), lambda bi, ti, ids: (ids[bi], 0, 0)),
        ],
        out_specs=pl.BlockSpec((1, _TB, HALF), lambda bi, ti, ids: (bi, ti, 0)),
    )
    return pl.pallas_call(
        _fc_body,
        grid_spec=grid_spec,
        out_shape=jax.ShapeDtypeStruct((B, T, HALF), jnp.float32),
    )(inst_ids, x, w.astype(jnp.bfloat16), b.reshape(1, HALF),
      inst_table.reshape(-1, 1, HALF))


def _combine_body(inst_ids, fc_ref, tok_ref, inst_ref, out_ref):
    out_ref[0, :, :HALF] = fc_ref[0]
    out_ref[0, :, HALF:] = tok_ref[0] + inst_ref[0]


def _tc_combine(fc, tok_sum, inst_ids, inst_table):
    grid_spec = pltpu.PrefetchScalarGridSpec(
        num_scalar_prefetch=1,
        grid=(B, T // _TB),
        in_specs=[
            pl.BlockSpec((1, _TB, HALF), lambda bi, ti, ids: (bi, ti, 0)),
            pl.BlockSpec((1, _TB, HALF), lambda bi, ti, ids: (bi, ti, 0)),
            pl.BlockSpec((1, 1, HALF), lambda bi, ti, ids: (ids[bi], 0, 0)),
        ],
        out_specs=pl.BlockSpec((1, _TB, MODEL_DIM), lambda bi, ti, ids: (bi, ti, 0)),
    )
    return pl.pallas_call(
        _combine_body,
        grid_spec=grid_spec,
        out_shape=jax.ShapeDtypeStruct((B, T, MODEL_DIM), jnp.float32),
    )(inst_ids, fc, tok_sum, inst_table.reshape(-1, 1, HALF))


def kernel(input, target_masked, target_inst_id, W_fc, b_fc, target_table, inst_table):
    offset = (jnp.arange(K, dtype=jnp.int32) * C)
    tok = target_masked + offset[None, :, None]
    tok_flat = tok.transpose(0, 2, 1).reshape(ITEMS * K)

    tok_sum = _sc_gather_sum()(tok_flat, target_table)
    tok_sum = tok_sum.reshape(B, T, HALF)

    fc = _tc_fc(input, W_fc, b_fc, target_inst_id, inst_table)
    return _tc_combine(fc, tok_sum, target_inst_id, inst_table)


# speedup vs baseline: 1.0975x; 1.0975x over previous
import functools

import jax
import jax.numpy as jnp
from jax import lax
from jax.experimental import pallas as pl
from jax.experimental.pallas import tpu as pltpu
from jax.experimental.pallas import tpu_sc as plsc

B, T, D = 4, 4096, 2048
K = 4
C = 2048
NUM_TOKENS = 8192
HALF = 1024
MODEL_DIM = 2048

ITEMS = B * T
NB = 8
ROWS = NB * K
LANES = 16

NC = 2
NS = 16


def _sc_gather_sum():
    nw = NC * NS
    ipw = ITEMS // nw
    nblk = ipw // NB
    mesh = plsc.VectorSubcoreMesh(
        core_axis_name="c", subcore_axis_name="s", num_cores=NC, num_subcores=NS
    )

    @functools.partial(
        pl.kernel,
        out_type=jax.ShapeDtypeStruct((ITEMS, HALF), jnp.float32),
        mesh=mesh,
        scratch_types=[
            pltpu.VMEM((ipw * K,), jnp.int32),
            pltpu.VMEM((2, ROWS, HALF), jnp.float32),
            pltpu.VMEM((2, NB, HALF), jnp.float32),
            pltpu.SemaphoreType.DMA,
            pltpu.SemaphoreType.DMA,
        ],
    )
    def sc_kernel(idx_hbm, table_hbm, out_hbm, idx_v, rows_v, out_v, gsem, osem):
        wid = lax.axis_index("s") * NC + lax.axis_index("c")
        base_item = wid * ipw

        pltpu.sync_copy(idx_hbm.at[pl.ds(base_item * K, ipw * K)], idx_v)

        def start_gather(blk, buf):
            pltpu.async_copy(
                table_hbm.at[idx_v.at[pl.ds(blk * ROWS, ROWS)]],
                rows_v.at[buf],
                gsem,
            )

        def wait_gather(buf):
            pltpu.make_async_copy(
                table_hbm.at[idx_v.at[pl.ds(0, ROWS)]], rows_v.at[buf], gsem
            ).wait()

        def wait_out(par):
            pltpu.make_async_copy(
                out_v.at[par], out_hbm.at[pl.ds(0, NB)], osem
            ).wait()

        start_gather(0, 0)

        def blk_pair(h, _):
            for par in (0, 1):
                blk = 2 * h + par
                nxt = jnp.minimum(blk + 1, nblk - 1)
                start_gather(nxt, 1 - par)
                wait_gather(par)

                @pl.when(h > 0)
                def _():
                    wait_out(par)

                @plsc.parallel_loop(0, HALF // LANES, 1, unroll=4)
                def _(j):
                    s = pl.ds(j * LANES, LANES)
                    for i in range(NB):
                        r0 = 4 * i
                        out_v[par, i, s] = (
                            (rows_v[par, r0, s] + rows_v[par, r0 + 1, s])
                            + (rows_v[par, r0 + 2, s] + rows_v[par, r0 + 3, s])
                        )
                pltpu.async_copy(
                    out_v.at[par],
                    out_hbm.at[pl.ds(base_item + blk * NB, NB)],
                    osem,
                )
            return 0

        lax.fori_loop(0, nblk // 2, blk_pair, 0)
        wait_gather(0)
        wait_out(0)
        wait_out(1)

    return sc_kernel


_TB = 512


def _tc_body(inst_ids, x_ref, w_ref, b_ref, inst_ref, tok_ref, out_ref):
    x = x_ref[0].astype(jnp.bfloat16)
    w = w_ref[...]
    acc = lax.dot_general(
        x, w, (((1,), (1,)), ((), ())),
        preferred_element_type=jnp.float32,
    )
    inst = inst_ref[0]
    out_ref[0, :, :HALF] = acc + b_ref[...] + inst
    out_ref[0, :, HALF:] = tok_ref[0] + inst


def _tc_assemble(x, w, b, inst_ids, inst_table, tok_sum):
    grid_spec = pltpu.PrefetchScalarGridSpec(
        num_scalar_prefetch=1,
        grid=(B, T // _TB),
        in_specs=[
            pl.BlockSpec((1, _TB, D), lambda bi, ti, ids: (bi, ti, 0)),
            pl.BlockSpec((HALF, D), lambda bi, ti, ids: (0, 0)),
            pl.BlockSpec((1, HALF), lambda bi, ti, ids: (0, 0)),
            pl.BlockSpec((1, 1, HALF), lambda bi, ti, ids: (ids[bi], 0, 0)),
            pl.BlockSpec((1, _TB, HALF), lambda bi, ti, ids: (bi, ti, 0)),
        ],
        out_specs=pl.BlockSpec((1, _TB, MODEL_DIM), lambda bi, ti, ids: (bi, ti, 0)),
    )
    return pl.pallas_call(
        _tc_body,
        grid_spec=grid_spec,
        out_shape=jax.ShapeDtypeStruct((B, T, MODEL_DIM), jnp.float32),
    )(inst_ids, x, w.astype(jnp.bfloat16), b.reshape(1, HALF),
      inst_table.reshape(-1, 1, HALF), tok_sum)


def kernel(input, target_masked, target_inst_id, W_fc, b_fc, target_table, inst_table):
    offset = (jnp.arange(K, dtype=jnp.int32) * C)
    tok = target_masked + offset[None, :, None]
    tok_flat = tok.transpose(0, 2, 1).reshape(ITEMS * K)

    tok_sum = _sc_gather_sum()(tok_flat, target_table)
    tok_sum = tok_sum.reshape(B, T, HALF)

    return _tc_assemble(input, W_fc, b_fc, target_inst_id, inst_table, tok_sum)


# speedup vs baseline: 1.1053x; 1.0070x over previous
import functools

import jax
import jax.numpy as jnp
from jax import lax
from jax.experimental import pallas as pl
from jax.experimental.pallas import tpu as pltpu
from jax.experimental.pallas import tpu_sc as plsc

B, T, D = 4, 4096, 2048
K = 4
C = 2048
NUM_TOKENS = 8192
HALF = 1024
MODEL_DIM = 2048

ITEMS = B * T
NB = 8
ROWS = NB * K
LANES = 16

NC = 2
NS = 16


def _sc_gather_sum():
    nw = NC * NS
    ipw = ITEMS // nw
    nblk = ipw // NB
    mesh = plsc.VectorSubcoreMesh(
        core_axis_name="c", subcore_axis_name="s", num_cores=NC, num_subcores=NS
    )

    @functools.partial(
        pl.kernel,
        out_type=jax.ShapeDtypeStruct((ITEMS, HALF), jnp.float32),
        mesh=mesh,
        scratch_types=[
            pltpu.VMEM((ipw * K,), jnp.int32),
            pltpu.VMEM((2, ROWS, HALF), jnp.float32),
            pltpu.VMEM((2, NB, HALF), jnp.float32),
            pltpu.SemaphoreType.DMA,
            pltpu.SemaphoreType.DMA,
        ],
    )
    def sc_kernel(idx_hbm, table_hbm, out_hbm, idx_v, rows_v, out_v, gsem, osem):
        wid = lax.axis_index("s") * NC + lax.axis_index("c")
        base_item = wid * ipw

        pltpu.sync_copy(idx_hbm.at[pl.ds(base_item * K, ipw * K)], idx_v)

        def start_gather(blk, buf):
            pltpu.async_copy(
                table_hbm.at[idx_v.at[pl.ds(blk * ROWS, ROWS)]],
                rows_v.at[buf],
                gsem,
            )

        def wait_gather(buf):
            pltpu.make_async_copy(
                table_hbm.at[idx_v.at[pl.ds(0, ROWS)]], rows_v.at[buf], gsem
            ).wait()

        def wait_out(par):
            pltpu.make_async_copy(
                out_v.at[par], out_hbm.at[pl.ds(0, NB)], osem
            ).wait()

        start_gather(0, 0)

        def blk_pair(h, _):
            for par in (0, 1):
                blk = 2 * h + par
                nxt = jnp.minimum(blk + 1, nblk - 1)
                start_gather(nxt, 1 - par)
                wait_gather(par)

                @pl.when(h > 0)
                def _():
                    wait_out(par)

                @plsc.parallel_loop(0, HALF // LANES, 1, unroll=4)
                def _(j):
                    s = pl.ds(j * LANES, LANES)
                    for i in range(NB):
                        r0 = 4 * i
                        out_v[par, i, s] = (
                            (rows_v[par, r0, s] + rows_v[par, r0 + 1, s])
                            + (rows_v[par, r0 + 2, s] + rows_v[par, r0 + 3, s])
                        )
                pltpu.async_copy(
                    out_v.at[par],
                    out_hbm.at[pl.ds(base_item + blk * NB, NB)],
                    osem,
                )
            return 0

        lax.fori_loop(0, nblk // 2, blk_pair, 0)
        wait_gather(0)
        wait_out(0)
        wait_out(1)

    return sc_kernel


_TB = 1024


def _tc_body(inst_ids, x_ref, w_ref, b_ref, inst_ref, tok_ref, out_ref):
    x = x_ref[0].astype(jnp.bfloat16)
    w = w_ref[...]
    acc = lax.dot_general(
        x, w, (((1,), (1,)), ((), ())),
        preferred_element_type=jnp.float32,
    )
    inst = inst_ref[0]
    out_ref[0, :, :HALF] = acc + b_ref[...] + inst
    out_ref[0, :, HALF:] = tok_ref[0] + inst


def _tc_assemble(x, w, b, inst_ids, inst_table, tok_sum):
    grid_spec = pltpu.PrefetchScalarGridSpec(
        num_scalar_prefetch=1,
        grid=(B, T // _TB),
        in_specs=[
            pl.BlockSpec((1, _TB, D), lambda bi, ti, ids: (bi, ti, 0)),
            pl.BlockSpec((HALF, D), lambda bi, ti, ids: (0, 0)),
            pl.BlockSpec((1, HALF), lambda bi, ti, ids: (0, 0)),
            pl.BlockSpec((1, 1, HALF), lambda bi, ti, ids: (ids[bi], 0, 0)),
            pl.BlockSpec((1, _TB, HALF), lambda bi, ti, ids: (bi, ti, 0)),
        ],
        out_specs=pl.BlockSpec((1, _TB, MODEL_DIM), lambda bi, ti, ids: (bi, ti, 0)),
    )
    return pl.pallas_call(
        _tc_body,
        grid_spec=grid_spec,
        out_shape=jax.ShapeDtypeStruct((B, T, MODEL_DIM), jnp.float32),
    )(inst_ids, x, w.astype(jnp.bfloat16), b.reshape(1, HALF),
      inst_table.reshape(-1, 1, HALF), tok_sum)


def kernel(input, target_masked, target_inst_id, W_fc, b_fc, target_table, inst_table):
    offset = (jnp.arange(K, dtype=jnp.int32) * C)
    tok = target_masked + offset[None, :, None]
    tok_flat = tok.transpose(0, 2, 1).reshape(ITEMS * K)

    tok_sum = _sc_gather_sum()(tok_flat, target_table)
    tok_sum = tok_sum.reshape(B, T, HALF)

    return _tc_assemble(input, W_fc, b_fc, target_inst_id, inst_table, tok_sum)
